# initial kernel scaffold (unmeasured)
import jax
import jax.numpy as jnp
from jax import lax
from jax.experimental import pallas as pl
from jax.experimental.pallas import tpu as pltpu

N_DEV = 4


def kernel(x, w_mat, scale_x, scale_w):
    k, _ = x.shape
    _, n = w_mat.shape
    m_per = k // N_DEV

    def body(x_ref, w_ref, sx_ref, sw_ref, out_ref,
             send_buf, recv_buf, send_sems, recv_sems):
        my = lax.axis_index("i")
        left = lax.rem(my + (N_DEV - 1), N_DEV)
        right = lax.rem(my + 1, N_DEV)

        barrier_sem = pltpu.get_barrier_semaphore()
        for nbr in (left, right):
            pl.semaphore_signal(barrier_sem, inc=1, device_id=(nbr,),
                                device_id_type=pl.DeviceIdType.MESH)
        pl.semaphore_wait(barrier_sem, 2)

        def chunk(c):
            xc = x_ref[pl.ds(c * m_per, m_per), :].astype(jnp.bfloat16)
            wb = w_ref[...].astype(jnp.bfloat16)
            return jnp.dot(xc, wb, preferred_element_type=jnp.float32)

        send_buf[0] = chunk(lax.rem(my + N_DEV - 1, N_DEV)).astype(jnp.bfloat16)

        acc = None
        for s in range(N_DEV - 1):
            rdma = pltpu.make_async_remote_copy(
                src_ref=send_buf.at[s],
                dst_ref=recv_buf.at[s],
                send_sem=send_sems.at[s],
                recv_sem=recv_sems.at[s],
                device_id=(right,),
                device_id_type=pl.DeviceIdType.MESH,
            )
            rdma.start()
            local = chunk(lax.rem(my + (2 - s) + N_DEV, N_DEV))
            rdma.wait()
            acc = local + recv_buf[s].astype(jnp.float32)
            if s < N_DEV - 2:
                send_buf[s + 1] = acc.astype(jnp.bfloat16)

        scale = sx_ref[0] * sw_ref[0]
        out_ref[...] = jnp.maximum(acc * scale, 0.0)

    return pl.pallas_call(
        body,
        out_shape=jax.ShapeDtypeStruct((m_per, n), jnp.float32),
        in_specs=[
            pl.BlockSpec(memory_space=pltpu.VMEM),
            pl.BlockSpec(memory_space=pltpu.VMEM),
            pl.BlockSpec(memory_space=pltpu.SMEM),
            pl.BlockSpec(memory_space=pltpu.SMEM),
        ],
        out_specs=pl.BlockSpec(memory_space=pltpu.VMEM),
        scratch_shapes=[
            pltpu.VMEM((N_DEV - 1, m_per, n), jnp.bfloat16),
            pltpu.VMEM((N_DEV - 1, m_per, n), jnp.bfloat16),
            pltpu.SemaphoreType.DMA((N_DEV - 1,)),
            pltpu.SemaphoreType.DMA((N_DEV - 1,)),
        ],
        compiler_params=pltpu.CompilerParams(collective_id=0),
    )(x, w_mat, scale_x, scale_w)


# baseline (device time: 191795 ns/iter reference)
import jax
import jax.numpy as jnp
from jax import lax
from jax.experimental import pallas as pl
from jax.experimental.pallas import tpu as pltpu

N_DEV = 4


def kernel(x, w_mat, scale_x, scale_w):
    k, _ = x.shape
    _, n = w_mat.shape
    m_per = k // N_DEV

    x = x.astype(jnp.bfloat16)
    w_mat = w_mat.astype(jnp.bfloat16)

    def body(x_ref, w_ref, sx_ref, sw_ref, out_ref,
             send_buf, recv_buf, send_sems, recv_sems):
        my = lax.axis_index("i")
        left = lax.rem(my + (N_DEV - 1), N_DEV)
        right = lax.rem(my + 1, N_DEV)

        barrier_sem = pltpu.get_barrier_semaphore()
        for nbr in (left, right):
            pl.semaphore_signal(barrier_sem, inc=1, device_id=(nbr,),
                                device_id_type=pl.DeviceIdType.MESH)
        pl.semaphore_wait(barrier_sem, 2)

        def chunk(c):
            xc = x_ref[pl.ds(c * m_per, m_per), :]
            return jnp.dot(xc, w_ref[...], preferred_element_type=jnp.float32)

        send_buf[0] = chunk(lax.rem(my + N_DEV - 1, N_DEV)).astype(jnp.bfloat16)

        acc = None
        for s in range(N_DEV - 1):
            rdma = pltpu.make_async_remote_copy(
                src_ref=send_buf.at[s],
                dst_ref=recv_buf.at[s],
                send_sem=send_sems.at[s],
                recv_sem=recv_sems.at[s],
                device_id=(right,),
                device_id_type=pl.DeviceIdType.MESH,
            )
            rdma.start()
            local = chunk(lax.rem(my + (2 - s) + N_DEV, N_DEV))
            rdma.wait()
            acc = local + recv_buf[s].astype(jnp.float32)
            if s < N_DEV - 2:
                send_buf[s + 1] = acc.astype(jnp.bfloat16)

        scale = sx_ref[0] * sw_ref[0]
        out_ref[...] = jnp.maximum(acc * scale, 0.0)

    return pl.pallas_call(
        body,
        out_shape=jax.ShapeDtypeStruct((m_per, n), jnp.float32),
        in_specs=[
            pl.BlockSpec(memory_space=pltpu.VMEM),
            pl.BlockSpec(memory_space=pltpu.VMEM),
            pl.BlockSpec(memory_space=pltpu.SMEM),
            pl.BlockSpec(memory_space=pltpu.SMEM),
        ],
        out_specs=pl.BlockSpec(memory_space=pltpu.VMEM),
        scratch_shapes=[
            pltpu.VMEM((N_DEV - 1, m_per, n), jnp.bfloat16),
            pltpu.VMEM((N_DEV - 1, m_per, n), jnp.bfloat16),
            pltpu.SemaphoreType.DMA((N_DEV - 1,)),
            pltpu.SemaphoreType.DMA((N_DEV - 1,)),
        ],
        compiler_params=pltpu.CompilerParams(
            collective_id=0,
            vmem_limit_bytes=110 * 1024 * 1024,
        ),
    )(x, w_mat, scale_x, scale_w)


# device time: 124398 ns/iter; 1.5418x vs baseline; 1.5418x over previous
import jax
import jax.numpy as jnp
from jax import lax
from jax.experimental import pallas as pl
from jax.experimental.pallas import tpu as pltpu

N_DEV = 4


def kernel(x, w_mat, scale_x, scale_w):
    k, _ = x.shape
    _, n = w_mat.shape
    m_per = k // N_DEV
    nh = n // 2

    x = x.astype(jnp.bfloat16)
    w_mat = w_mat.astype(jnp.bfloat16)

    def body(x_ref, w_ref, sx_ref, sw_ref, out_ref,
             send_r, recv_r, send_l, recv_l,
             send_sems_r, recv_sems_r, send_sems_l, recv_sems_l):
        my = lax.axis_index("i")
        left = lax.rem(my + (N_DEV - 1), N_DEV)
        right = lax.rem(my + 1, N_DEV)

        barrier_sem = pltpu.get_barrier_semaphore()
        for nbr in (left, right):
            pl.semaphore_signal(barrier_sem, inc=1, device_id=(nbr,),
                                device_id_type=pl.DeviceIdType.MESH)
        pl.semaphore_wait(barrier_sem, 2)

        def chunk(c, col0, ncol):
            xc = x_ref[pl.ds(c * m_per, m_per), :]
            return jnp.dot(xc, w_ref[:, col0:col0 + ncol],
                           preferred_element_type=jnp.float32)

        send_r[0] = chunk(lax.rem(my + 3, N_DEV), 0, nh).astype(jnp.bfloat16)
        send_l[0] = chunk(lax.rem(my + 1, N_DEV), nh, nh).astype(jnp.bfloat16)

        acc_r = acc_l = None
        for s in range(N_DEV - 1):
            rdma_r = pltpu.make_async_remote_copy(
                src_ref=send_r.at[s], dst_ref=recv_r.at[s],
                send_sem=send_sems_r.at[s], recv_sem=recv_sems_r.at[s],
                device_id=(right,), device_id_type=pl.DeviceIdType.MESH,
            )
            rdma_l = pltpu.make_async_remote_copy(
                src_ref=send_l.at[s], dst_ref=recv_l.at[s],
                send_sem=send_sems_l.at[s], recv_sem=recv_sems_l.at[s],
                device_id=(left,), device_id_type=pl.DeviceIdType.MESH,
            )
            rdma_r.start()
            rdma_l.start()
            local_r = chunk(lax.rem(my + (2 - s) + N_DEV, N_DEV), 0, nh)
            local_l = chunk(lax.rem(my + 2 + s, N_DEV), nh, nh)
            rdma_r.wait()
            rdma_l.wait()
            acc_r = local_r + recv_r[s].astype(jnp.float32)
            acc_l = local_l + recv_l[s].astype(jnp.float32)
            if s < N_DEV - 2:
                send_r[s + 1] = acc_r.astype(jnp.bfloat16)
                send_l[s + 1] = acc_l.astype(jnp.bfloat16)

        scale = sx_ref[0] * sw_ref[0]
        out_ref[:, 0:nh] = jnp.maximum(acc_r * scale, 0.0)
        out_ref[:, nh:n] = jnp.maximum(acc_l * scale, 0.0)

    return pl.pallas_call(
        body,
        out_shape=jax.ShapeDtypeStruct((m_per, n), jnp.float32),
        in_specs=[
            pl.BlockSpec(memory_space=pltpu.VMEM),
            pl.BlockSpec(memory_space=pltpu.VMEM),
            pl.BlockSpec(memory_space=pltpu.SMEM),
            pl.BlockSpec(memory_space=pltpu.SMEM),
        ],
        out_specs=pl.BlockSpec(memory_space=pltpu.VMEM),
        scratch_shapes=[
            pltpu.VMEM((N_DEV - 1, m_per, nh), jnp.bfloat16),
            pltpu.VMEM((N_DEV - 1, m_per, nh), jnp.bfloat16),
            pltpu.VMEM((N_DEV - 1, m_per, nh), jnp.bfloat16),
            pltpu.VMEM((N_DEV - 1, m_per, nh), jnp.bfloat16),
            pltpu.SemaphoreType.DMA((N_DEV - 1,)),
            pltpu.SemaphoreType.DMA((N_DEV - 1,)),
            pltpu.SemaphoreType.DMA((N_DEV - 1,)),
            pltpu.SemaphoreType.DMA((N_DEV - 1,)),
        ],
        compiler_params=pltpu.CompilerParams(
            collective_id=0,
            vmem_limit_bytes=110 * 1024 * 1024,
        ),
    )(x, w_mat, scale_x, scale_w)


# device time: 107370 ns/iter; 1.7863x vs baseline; 1.1586x over previous
import jax
import jax.numpy as jnp
from jax import lax
from jax.experimental import pallas as pl
from jax.experimental.pallas import tpu as pltpu

N_DEV = 4


def kernel(x, w_mat, scale_x, scale_w):
    k, _ = x.shape
    _, n = w_mat.shape
    m_per = k // N_DEV
    nh = n // 2

    w_mat = w_mat.astype(jnp.bfloat16)

    def body(x_ref, w_ref, sx_ref, sw_ref, out_ref,
             xf, send_r, recv_r, send_l, recv_l,
             load_sems, send_sems_r, recv_sems_r, send_sems_l, recv_sems_l):
        my = lax.axis_index("i")
        left = lax.rem(my + (N_DEV - 1), N_DEV)
        right = lax.rem(my + 1, N_DEV)

        c_a = lax.rem(my + 3, N_DEV)
        c_b = lax.rem(my + 1, N_DEV)
        c_c = lax.rem(my + 2, N_DEV)
        c_d = my

        def load(slot, c):
            cp = pltpu.make_async_copy(
                x_ref.at[pl.ds(c * m_per, m_per), :],
                xf.at[slot],
                load_sems.at[slot],
            )
            cp.start()
            return cp

        ld_a = load(0, c_a)
        ld_b = load(1, c_b)

        barrier_sem = pltpu.get_barrier_semaphore()
        for nbr in (left, right):
            pl.semaphore_signal(barrier_sem, inc=1, device_id=(nbr,),
                                device_id_type=pl.DeviceIdType.MESH)
        pl.semaphore_wait(barrier_sem, 2)

        rd_r = [
            pltpu.make_async_remote_copy(
                src_ref=send_r.at[s], dst_ref=recv_r.at[s],
                send_sem=send_sems_r.at[s], recv_sem=recv_sems_r.at[s],
                device_id=(right,), device_id_type=pl.DeviceIdType.MESH,
            )
            for s in range(N_DEV - 1)
        ]
        rd_l = [
            pltpu.make_async_remote_copy(
                src_ref=send_l.at[s], dst_ref=recv_l.at[s],
                send_sem=send_sems_l.at[s], recv_sem=recv_sems_l.at[s],
                device_id=(left,), device_id_type=pl.DeviceIdType.MESH,
            )
            for s in range(N_DEV - 1)
        ]

        def half_gemm(slot, col0):
            xc = xf[slot].astype(jnp.bfloat16)
            return jnp.dot(xc, w_ref[:, col0:col0 + nh],
                           preferred_element_type=jnp.float32)

        ld_a.wait()
        send_r[0] = half_gemm(0, 0).astype(jnp.bfloat16)
        rd_r[0].start()
        ld_b.wait()
        send_l[0] = half_gemm(1, nh).astype(jnp.bfloat16)
        rd_l[0].start()

        ld_c = load(2, c_c)
        ld_d = load(3, c_d)

        acc_r = acc_l = None
        for s in range(N_DEV - 1):
            if s == 0:
                ld_c.wait()
                local_r = half_gemm(2, 0)
                local_l = half_gemm(2, nh)
            elif s == 1:
                local_r = half_gemm(1, 0)
                local_l = half_gemm(0, nh)
            else:
                ld_d.wait()
                local_r = half_gemm(3, 0)
                local_l = half_gemm(3, nh)

            rd_r[s].wait_recv()
            acc_r = local_r + recv_r[s].astype(jnp.float32)
            if s < N_DEV - 2:
                send_r[s + 1] = acc_r.astype(jnp.bfloat16)
                rd_r[s + 1].start()
            rd_l[s].wait_recv()
            acc_l = local_l + recv_l[s].astype(jnp.float32)
            if s < N_DEV - 2:
                send_l[s + 1] = acc_l.astype(jnp.bfloat16)
                rd_l[s + 1].start()

        scale = sx_ref[0] * sw_ref[0]
        out_ref[:, 0:nh] = jnp.maximum(acc_r * scale, 0.0)
        out_ref[:, nh:n] = jnp.maximum(acc_l * scale, 0.0)

        for s in range(N_DEV - 1):
            rd_r[s].wait_send()
            rd_l[s].wait_send()

    return pl.pallas_call(
        body,
        out_shape=jax.ShapeDtypeStruct((m_per, n), jnp.float32),
        in_specs=[
            pl.BlockSpec(memory_space=pl.ANY),
            pl.BlockSpec(memory_space=pltpu.VMEM),
            pl.BlockSpec(memory_space=pltpu.SMEM),
            pl.BlockSpec(memory_space=pltpu.SMEM),
        ],
        out_specs=pl.BlockSpec(memory_space=pltpu.VMEM),
        scratch_shapes=[
            pltpu.VMEM((N_DEV, m_per, k // N_DEV), jnp.float32),
            pltpu.VMEM((N_DEV - 1, m_per, nh), jnp.bfloat16),
            pltpu.VMEM((N_DEV - 1, m_per, nh), jnp.bfloat16),
            pltpu.VMEM((N_DEV - 1, m_per, nh), jnp.bfloat16),
            pltpu.VMEM((N_DEV - 1, m_per, nh), jnp.bfloat16),
            pltpu.SemaphoreType.DMA((N_DEV,)),
            pltpu.SemaphoreType.DMA((N_DEV - 1,)),
            pltpu.SemaphoreType.DMA((N_DEV - 1,)),
            pltpu.SemaphoreType.DMA((N_DEV - 1,)),
            pltpu.SemaphoreType.DMA((N_DEV - 1,)),
        ],
        compiler_params=pltpu.CompilerParams(
            collective_id=0,
            vmem_limit_bytes=110 * 1024 * 1024,
        ),
    )(x, w_mat, scale_x, scale_w)


# device time: 96696 ns/iter; 1.9835x vs baseline; 1.1104x over previous
import jax
import jax.numpy as jnp
from jax import lax
from jax.experimental import pallas as pl
from jax.experimental.pallas import tpu as pltpu

N_DEV = 4
SUB = 2


def kernel(x, w_mat, scale_x, scale_w):
    k, kloc = x.shape
    _, n = w_mat.shape
    m_per = k // N_DEV
    nh = n // 2
    mb = m_per // SUB

    w_mat = w_mat.astype(jnp.bfloat16)

    def body(x_ref, w_ref, sx_ref, sw_ref, out_ref,
             xf, send_r, recv_r, send_l, recv_l,
             load_sems, send_sems_r, recv_sems_r, send_sems_l, recv_sems_l):
        my = lax.axis_index("i")
        left = lax.rem(my + (N_DEV - 1), N_DEV)
        right = lax.rem(my + 1, N_DEV)

        c_a = lax.rem(my + 3, N_DEV)
        c_b = lax.rem(my + 1, N_DEV)
        c_c = lax.rem(my + 2, N_DEV)
        c_d = my

        def load(slot, c):
            cp = pltpu.make_async_copy(
                x_ref.at[pl.ds(c * m_per, m_per), :],
                xf.at[slot],
                load_sems.at[slot],
            )
            cp.start()
            return cp

        ld_a = load(0, c_a)
        ld_b = load(1, c_b)

        barrier_sem = pltpu.get_barrier_semaphore()
        for nbr in (left, right):
            pl.semaphore_signal(barrier_sem, inc=1, device_id=(nbr,),
                                device_id_type=pl.DeviceIdType.MESH)
        pl.semaphore_wait(barrier_sem, 2)

        def rd(buf_s, buf_d, sems_s, sems_d, dev, s, b):
            return pltpu.make_async_remote_copy(
                src_ref=buf_s.at[s, pl.ds(b * mb, mb), :],
                dst_ref=buf_d.at[s, pl.ds(b * mb, mb), :],
                send_sem=sems_s.at[s, b], recv_sem=sems_d.at[s, b],
                device_id=(dev,), device_id_type=pl.DeviceIdType.MESH,
            )

        rd_r = [[rd(send_r, recv_r, send_sems_r, recv_sems_r, right, s, b)
                 for b in range(SUB)] for s in range(N_DEV - 1)]
        rd_l = [[rd(send_l, recv_l, send_sems_l, recv_sems_l, left, s, b)
                 for b in range(SUB)] for s in range(N_DEV - 1)]

        def sub_gemm(slot, b, col0, out_dtype):
            xc = xf[slot, pl.ds(b * mb, mb), :].astype(jnp.bfloat16)
            r = jnp.dot(xc, w_ref[:, col0:col0 + nh],
                        preferred_element_type=jnp.float32)
            return r if out_dtype == jnp.float32 else r.astype(out_dtype)

        ld_a.wait()
        ld_b.wait()
        for b in range(SUB):
            send_r[0, pl.ds(b * mb, mb), :] = sub_gemm(0, b, 0, jnp.bfloat16)
            rd_r[0][b].start()
            send_l[0, pl.ds(b * mb, mb), :] = sub_gemm(1, b, nh, jnp.bfloat16)
            rd_l[0][b].start()

        ld_c = load(2, c_c)
        ld_d = load(3, c_d)

        acc_r = [None] * SUB
        acc_l = [None] * SUB
        for s in range(N_DEV - 1):
            last = s == N_DEV - 2
            acc_dt = jnp.float32 if last else jnp.bfloat16
            if s == 0:
                ld_c.wait()
                slot_r = slot_l = 2
            elif s == 1:
                slot_r, slot_l = 1, 0
            else:
                ld_d.wait()
                slot_r = slot_l = 3

            local_r = [sub_gemm(slot_r, b, 0, acc_dt) for b in range(SUB)]
            local_l = [sub_gemm(slot_l, b, nh, acc_dt) for b in range(SUB)]

            for b in range(SUB):
                rows = pl.ds(b * mb, mb)
                rd_r[s][b].wait_recv()
                rv = recv_r[s, rows, :]
                acc_r[b] = local_r[b] + (rv.astype(jnp.float32) if last else rv)
                if not last:
                    send_r[s + 1, rows, :] = acc_r[b]
                    rd_r[s + 1][b].start()
                rd_l[s][b].wait_recv()
                lv = recv_l[s, rows, :]
                acc_l[b] = local_l[b] + (lv.astype(jnp.float32) if last else lv)
                if not last:
                    send_l[s + 1, rows, :] = acc_l[b]
                    rd_l[s + 1][b].start()

        scale = sx_ref[0] * sw_ref[0]
        for b in range(SUB):
            rows = pl.ds(b * mb, mb)
            out_ref[rows, 0:nh] = jnp.maximum(acc_r[b] * scale, 0.0)
            out_ref[rows, nh:n] = jnp.maximum(acc_l[b] * scale, 0.0)

        for s in range(N_DEV - 1):
            for b in range(SUB):
                rd_r[s][b].wait_send()
                rd_l[s][b].wait_send()

    return pl.pallas_call(
        body,
        out_shape=jax.ShapeDtypeStruct((m_per, n), jnp.float32),
        in_specs=[
            pl.BlockSpec(memory_space=pl.ANY),
            pl.BlockSpec(memory_space=pltpu.VMEM),
            pl.BlockSpec(memory_space=pltpu.SMEM),
            pl.BlockSpec(memory_space=pltpu.SMEM),
        ],
        out_specs=pl.BlockSpec(memory_space=pltpu.VMEM),
        scratch_shapes=[
            pltpu.VMEM((N_DEV, m_per, kloc), jnp.float32),
            pltpu.VMEM((N_DEV - 1, m_per, nh), jnp.bfloat16),
            pltpu.VMEM((N_DEV - 1, m_per, nh), jnp.bfloat16),
            pltpu.VMEM((N_DEV - 1, m_per, nh), jnp.bfloat16),
            pltpu.VMEM((N_DEV - 1, m_per, nh), jnp.bfloat16),
            pltpu.SemaphoreType.DMA((N_DEV,)),
            pltpu.SemaphoreType.DMA((N_DEV - 1, SUB)),
            pltpu.SemaphoreType.DMA((N_DEV - 1, SUB)),
            pltpu.SemaphoreType.DMA((N_DEV - 1, SUB)),
            pltpu.SemaphoreType.DMA((N_DEV - 1, SUB)),
        ],
        compiler_params=pltpu.CompilerParams(
            collective_id=0,
            vmem_limit_bytes=110 * 1024 * 1024,
        ),
    )(x, w_mat, scale_x, scale_w)


# device time: 92481 ns/iter; 2.0739x vs baseline; 1.0456x over previous
import jax
import jax.numpy as jnp
from jax import lax
from jax.experimental import pallas as pl
from jax.experimental.pallas import tpu as pltpu

N_DEV = 4
SUB = 2


def kernel(x, w_mat, scale_x, scale_w):
    k, kloc = x.shape
    _, n = w_mat.shape
    m_per = k // N_DEV
    nh = n // 2
    mb = m_per // SUB

    def body(x_ref, w_ref, sx_ref, sw_ref, out_ref,
             xf, w_bf, send_r, recv_r, send_l, recv_l,
             load_sems, out_sems,
             send_sems_r, recv_sems_r, send_sems_l, recv_sems_l):
        my = lax.axis_index("i")
        left = lax.rem(my + (N_DEV - 1), N_DEV)
        right = lax.rem(my + 1, N_DEV)

        c_a = lax.rem(my + 3, N_DEV)
        c_b = lax.rem(my + 1, N_DEV)
        c_c = lax.rem(my + 2, N_DEV)
        c_d = my

        def load(slot, c):
            cp = pltpu.make_async_copy(
                x_ref.at[pl.ds(c * m_per, m_per), :],
                xf.at[slot],
                load_sems.at[slot],
            )
            cp.start()
            return cp

        ld_a = load(0, c_a)
        ld_b = load(1, c_b)
        w_bf[...] = w_ref[...].astype(jnp.bfloat16)

        barrier_sem = pltpu.get_barrier_semaphore()
        for nbr in (left, right):
            pl.semaphore_signal(barrier_sem, inc=1, device_id=(nbr,),
                                device_id_type=pl.DeviceIdType.MESH)
        pl.semaphore_wait(barrier_sem, 2)

        def rd(buf_s, buf_d, sems_s, sems_d, dev, s, b):
            return pltpu.make_async_remote_copy(
                src_ref=buf_s.at[s, pl.ds(b * mb, mb), :],
                dst_ref=buf_d.at[s, pl.ds(b * mb, mb), :],
                send_sem=sems_s.at[s, b], recv_sem=sems_d.at[s, b],
                device_id=(dev,), device_id_type=pl.DeviceIdType.MESH,
            )

        rd_r = [[rd(send_r, recv_r, send_sems_r, recv_sems_r, right, s, b)
                 for b in range(SUB)] for s in range(N_DEV - 1)]
        rd_l = [[rd(send_l, recv_l, send_sems_l, recv_sems_l, left, s, b)
                 for b in range(SUB)] for s in range(N_DEV - 1)]

        def sub_gemm(slot, b, col0, out_dtype):
            xc = xf[slot, pl.ds(b * mb, mb), :].astype(jnp.bfloat16)
            r = jnp.dot(xc, w_bf[:, col0:col0 + nh],
                        preferred_element_type=jnp.float32)
            return r if out_dtype == jnp.float32 else r.astype(out_dtype)

        ld_a.wait()
        ld_b.wait()
        for b in range(SUB):
            send_r[0, pl.ds(b * mb, mb), :] = sub_gemm(0, b, 0, jnp.bfloat16)
            rd_r[0][b].start()
            send_l[0, pl.ds(b * mb, mb), :] = sub_gemm(1, b, nh, jnp.bfloat16)
            rd_l[0][b].start()

        ld_c = load(2, c_c)
        ld_d = load(3, c_d)

        scale = sx_ref[0] * sw_ref[0]
        for s in range(N_DEV - 1):
            last = s == N_DEV - 2
            acc_dt = jnp.float32 if last else jnp.bfloat16
            if s == 0:
                ld_c.wait()
                slot_r = slot_l = 2
            elif s == 1:
                slot_r, slot_l = 1, 0
            else:
                ld_d.wait()
                slot_r = slot_l = 3

            local_r = [sub_gemm(slot_r, b, 0, acc_dt) for b in range(SUB)]
            local_l = [sub_gemm(slot_l, b, nh, acc_dt) for b in range(SUB)]

            for b in range(SUB):
                rows = pl.ds(b * mb, mb)
                rd_r[s][b].wait_recv()
                rv = recv_r[s, rows, :]
                if last:
                    y = local_r[b] + rv.astype(jnp.float32)
                    xf[0, rows, :] = jnp.maximum(y * scale, 0.0)
                else:
                    send_r[s + 1, rows, :] = local_r[b] + rv
                    rd_r[s + 1][b].start()
                rd_l[s][b].wait_recv()
                lv = recv_l[s, rows, :]
                if last:
                    y = local_l[b] + lv.astype(jnp.float32)
                    xf[1, rows, :] = jnp.maximum(y * scale, 0.0)
                else:
                    send_l[s + 1, rows, :] = local_l[b] + lv
                    rd_l[s + 1][b].start()

        st_r = pltpu.make_async_copy(xf.at[0], out_ref.at[:, 0:nh],
                                     out_sems.at[0])
        st_r.start()
        st_l = pltpu.make_async_copy(xf.at[1], out_ref.at[:, nh:n],
                                     out_sems.at[1])
        st_l.start()
        st_r.wait()
        st_l.wait()

        for s in range(N_DEV - 1):
            for b in range(SUB):
                rd_r[s][b].wait_send()
                rd_l[s][b].wait_send()

    return pl.pallas_call(
        body,
        out_shape=jax.ShapeDtypeStruct((m_per, n), jnp.float32),
        in_specs=[
            pl.BlockSpec(memory_space=pl.ANY),
            pl.BlockSpec(memory_space=pltpu.VMEM),
            pl.BlockSpec(memory_space=pltpu.SMEM),
            pl.BlockSpec(memory_space=pltpu.SMEM),
        ],
        out_specs=pl.BlockSpec(memory_space=pl.ANY),
        scratch_shapes=[
            pltpu.VMEM((N_DEV, m_per, kloc), jnp.float32),
            pltpu.VMEM((m_per, n), jnp.bfloat16),
            pltpu.VMEM((N_DEV - 1, m_per, nh), jnp.bfloat16),
            pltpu.VMEM((N_DEV - 1, m_per, nh), jnp.bfloat16),
            pltpu.VMEM((N_DEV - 1, m_per, nh), jnp.bfloat16),
            pltpu.VMEM((N_DEV - 1, m_per, nh), jnp.bfloat16),
            pltpu.SemaphoreType.DMA((N_DEV,)),
            pltpu.SemaphoreType.DMA((2,)),
            pltpu.SemaphoreType.DMA((N_DEV - 1, SUB)),
            pltpu.SemaphoreType.DMA((N_DEV - 1, SUB)),
            pltpu.SemaphoreType.DMA((N_DEV - 1, SUB)),
            pltpu.SemaphoreType.DMA((N_DEV - 1, SUB)),
        ],
        compiler_params=pltpu.CompilerParams(
            collective_id=0,
            vmem_limit_bytes=110 * 1024 * 1024,
        ),
    )(x, w_mat, scale_x, scale_w)


# device time: 92059 ns/iter; 2.0834x vs baseline; 1.0046x over previous
import jax
import jax.numpy as jnp
from jax import lax
from jax.experimental import pallas as pl
from jax.experimental.pallas import tpu as pltpu

N_DEV = 4
SUB = 2


def kernel(x, w_mat, scale_x, scale_w):
    k, kloc = x.shape
    _, n = w_mat.shape
    m_per = k // N_DEV
    nh = n // 2
    mb = m_per // SUB

    def body(x_ref, w_ref, sx_ref, sw_ref, out_ref,
             xf, w_bf, send_r, recv_r, send_l, recv_l,
             load_sems, out_sems,
             send_sems_r, recv_sems_r, send_sems_l, recv_sems_l):
        my = lax.axis_index("i")
        left = lax.rem(my + (N_DEV - 1), N_DEV)
        right = lax.rem(my + 1, N_DEV)

        c_a = lax.rem(my + 3, N_DEV)
        c_b = lax.rem(my + 1, N_DEV)
        c_c = lax.rem(my + 2, N_DEV)
        c_d = my

        def load(slot, c):
            cp = pltpu.make_async_copy(
                x_ref.at[pl.ds(c * m_per, m_per), :],
                xf.at[slot],
                load_sems.at[slot],
            )
            cp.start()
            return cp

        ld_a = load(0, c_a)
        ld_b = load(1, c_b)

        barrier_sem = pltpu.get_barrier_semaphore()
        for nbr in (left, right):
            pl.semaphore_signal(barrier_sem, inc=1, device_id=(nbr,),
                                device_id_type=pl.DeviceIdType.MESH)
        w_bf[...] = w_ref[...].astype(jnp.bfloat16)
        pl.semaphore_wait(barrier_sem, 2)

        def rd(buf_s, buf_d, sems_s, sems_d, dev, s, b):
            return pltpu.make_async_remote_copy(
                src_ref=buf_s.at[s, pl.ds(b * mb, mb), :],
                dst_ref=buf_d.at[s, pl.ds(b * mb, mb), :],
                send_sem=sems_s.at[s, b], recv_sem=sems_d.at[s, b],
                device_id=(dev,), device_id_type=pl.DeviceIdType.MESH,
            )

        rd_r = [[rd(send_r, recv_r, send_sems_r, recv_sems_r, right, s, b)
                 for b in range(SUB)] for s in range(N_DEV - 1)]
        rd_l = [[rd(send_l, recv_l, send_sems_l, recv_sems_l, left, s, b)
                 for b in range(SUB)] for s in range(N_DEV - 1)]

        def sub_gemm(slot, b, col0, out_dtype):
            xc = xf[slot, pl.ds(b * mb, mb), :].astype(jnp.bfloat16)
            r = jnp.dot(xc, w_bf[:, col0:col0 + nh],
                        preferred_element_type=jnp.float32)
            return r if out_dtype == jnp.float32 else r.astype(out_dtype)

        ld_a.wait()
        ld_b.wait()
        for b in range(SUB):
            send_r[0, pl.ds(b * mb, mb), :] = sub_gemm(0, b, 0, jnp.bfloat16)
            rd_r[0][b].start()
            send_l[0, pl.ds(b * mb, mb), :] = sub_gemm(1, b, nh, jnp.bfloat16)
            rd_l[0][b].start()

        ld_c = load(2, c_c)
        ld_d = load(3, c_d)

        scale = sx_ref[0] * sw_ref[0]
        for s in range(N_DEV - 1):
            last = s == N_DEV - 2
            acc_dt = jnp.float32 if last else jnp.bfloat16
            if s == 0:
                ld_c.wait()
                slot_r = slot_l = 2
            elif s == 1:
                slot_r, slot_l = 1, 0
            else:
                ld_d.wait()
                slot_r = slot_l = 3

            local_r = [sub_gemm(slot_r, b, 0, acc_dt) for b in range(SUB)]
            local_l = [sub_gemm(slot_l, b, nh, acc_dt) for b in range(SUB)]

            for b in range(SUB):
                rows = pl.ds(b * mb, mb)
                rd_r[s][b].wait_recv()
                rv = recv_r[s, rows, :]
                if last:
                    y = local_r[b] + rv.astype(jnp.float32)
                    xf[0, rows, :] = jnp.maximum(y * scale, 0.0)
                else:
                    send_r[s + 1, rows, :] = local_r[b] + rv
                    rd_r[s + 1][b].start()
                rd_l[s][b].wait_recv()
                lv = recv_l[s, rows, :]
                if last:
                    y = local_l[b] + lv.astype(jnp.float32)
                    xf[1, rows, :] = jnp.maximum(y * scale, 0.0)
                else:
                    send_l[s + 1, rows, :] = local_l[b] + lv
                    rd_l[s + 1][b].start()

        st_r = pltpu.make_async_copy(xf.at[0], out_ref.at[:, 0:nh],
                                     out_sems.at[0])
        st_r.start()
        st_l = pltpu.make_async_copy(xf.at[1], out_ref.at[:, nh:n],
                                     out_sems.at[1])
        st_l.start()
        st_r.wait()
        st_l.wait()

        for s in range(N_DEV - 1):
            for b in range(SUB):
                rd_r[s][b].wait_send()
                rd_l[s][b].wait_send()

    return pl.pallas_call(
        body,
        out_shape=jax.ShapeDtypeStruct((m_per, n), jnp.float32),
        in_specs=[
            pl.BlockSpec(memory_space=pl.ANY),
            pl.BlockSpec(memory_space=pltpu.VMEM),
            pl.BlockSpec(memory_space=pltpu.SMEM),
            pl.BlockSpec(memory_space=pltpu.SMEM),
        ],
        out_specs=pl.BlockSpec(memory_space=pl.ANY),
        scratch_shapes=[
            pltpu.VMEM((N_DEV, m_per, kloc), jnp.float32),
            pltpu.VMEM((m_per, n), jnp.bfloat16),
            pltpu.VMEM((N_DEV - 1, m_per, nh), jnp.bfloat16),
            pltpu.VMEM((N_DEV - 1, m_per, nh), jnp.bfloat16),
            pltpu.VMEM((N_DEV - 1, m_per, nh), jnp.bfloat16),
            pltpu.VMEM((N_DEV - 1, m_per, nh), jnp.bfloat16),
            pltpu.SemaphoreType.DMA((N_DEV,)),
            pltpu.SemaphoreType.DMA((2,)),
            pltpu.SemaphoreType.DMA((N_DEV - 1, SUB)),
            pltpu.SemaphoreType.DMA((N_DEV - 1, SUB)),
            pltpu.SemaphoreType.DMA((N_DEV - 1, SUB)),
            pltpu.SemaphoreType.DMA((N_DEV - 1, SUB)),
        ],
        compiler_params=pltpu.CompilerParams(
            collective_id=0,
            vmem_limit_bytes=110 * 1024 * 1024,
        ),
    )(x, w_mat, scale_x, scale_w)
